# baseline (speedup 1.0000x reference)
import functools

import jax
import jax.numpy as jnp
from jax.experimental import pallas as pl
from jax.experimental.pallas import tpu as pltpu

_BINS = 64
_EPS = 1e-10
_LANES = 1024
_SUB_R = 16


def _loss_kernel(pa_ref, ta_ref, f_ref, w_ref, out_ref, mm_ref, acc_ref,
                 ohf_ref, ohw_ref, *, nchunk, chunk_r):
    phase = pl.program_id(1)
    c = pl.program_id(2)

    @pl.when(jnp.logical_and(phase == 0, c == 0))
    def _init_mm():
        mm_ref[0] = jnp.float32(jnp.inf)
        mm_ref[1] = jnp.float32(-jnp.inf)
        mm_ref[2] = jnp.float32(jnp.inf)
        mm_ref[3] = jnp.float32(-jnp.inf)

    @pl.when(phase == 0)
    def _minmax():
        fb = f_ref[0]
        wb = w_ref[0]
        mm_ref[0] = jnp.minimum(mm_ref[0], jnp.min(fb))
        mm_ref[1] = jnp.maximum(mm_ref[1], jnp.max(fb))
        mm_ref[2] = jnp.minimum(mm_ref[2], jnp.min(wb))
        mm_ref[3] = jnp.maximum(mm_ref[3], jnp.max(wb))

    @pl.when(phase == 1)
    def _hist():
        @pl.when(c == 0)
        def _zero_acc():
            acc_ref[...] = jnp.zeros_like(acc_ref)

        fmin = mm_ref[0]
        wmin = mm_ref[2]
        inv_f = 1.0 / (mm_ref[1] - fmin + _EPS)
        inv_w = 1.0 / (mm_ref[3] - wmin + _EPS)

        def body(i, _):
            f8 = f_ref[0, pl.ds(i * _SUB_R, _SUB_R), :]
            w8 = w_ref[0, pl.ds(i * _SUB_R, _SUB_R), :]
            fi8 = jnp.clip(jnp.floor((f8 - fmin) * inv_f * (_BINS - 1)),
                           0.0, _BINS - 1.0).astype(jnp.bfloat16)
            wi8 = jnp.clip(jnp.floor((w8 - wmin) * inv_w * (_BINS - 1)),
                           0.0, _BINS - 1.0).astype(jnp.bfloat16)
            for b in range(_BINS):
                bv = jnp.bfloat16(b)
                ohf_ref[b] = jnp.where(fi8 == bv, jnp.bfloat16(1),
                                       jnp.bfloat16(0))
                ohw_ref[b] = jnp.where(wi8 == bv, jnp.bfloat16(1),
                                       jnp.bfloat16(0))
            parts = []
            for g in range(_SUB_R):
                parts.append(jax.lax.dot_general(
                    ohf_ref[:, g, :], ohw_ref[:, g, :],
                    (((1,), (1,)), ((), ())),
                    preferred_element_type=jnp.float32))
            while len(parts) > 1:
                parts = [a + b for a, b in zip(parts[::2], parts[1::2])]
            acc_ref[...] += parts[0]
            return 0

        jax.lax.fori_loop(0, chunk_r // _SUB_R, body, 0)

        @pl.when(c == nchunk - 1)
        def _finalize():
            counts = acc_ref[...]
            total = jnp.sum(counts)
            inv_n = 1.0 / (total + _EPS)
            joint = counts * inv_n
            p_f = jnp.sum(joint, axis=1, keepdims=True)
            p_w = jnp.sum(joint, axis=0, keepdims=True)
            h_f = -jnp.sum(p_f * jnp.log(p_f + _EPS))
            h_w = -jnp.sum(p_w * jnp.log(p_w + _EPS))
            h_j = -jnp.sum(joint * jnp.log(joint + _EPS))
            mi = h_f + h_w - h_j
            nmi = 2.0 * mi / (h_f + h_w + _EPS)
            d = pa_ref[0] - ta_ref[0]
            sumsq = jnp.sum(d * d)
            lane = jax.lax.broadcasted_iota(jnp.int32, (1, 128), 1)
            out_ref[0] = jnp.where(lane == 0, sumsq,
                                   jnp.where(lane == 1, nmi, 0.0))


def _choose_chunk(nr):
    for cr in (128, 64, 32, 16, 8):
        if nr % cr == 0:
            return cr
    return nr


@jax.jit
def kernel(predicted_affine, true_affine, fixed, warped):
    b = fixed.shape[0]
    n = fixed.shape[2] * fixed.shape[3] * fixed.shape[4]
    nr = n // _LANES
    chunk_r = _choose_chunk(nr)
    nchunk = nr // chunk_r

    f = fixed[:, 0].reshape(b, nr, _LANES)
    w = warped[:, 0].reshape(b, nr, _LANES)
    na = predicted_affine.shape[1]
    pa = jnp.pad(predicted_affine[:, None, :], ((0, 0), (0, 0), (0, 128 - na)))
    ta = jnp.pad(true_affine[:, None, :], ((0, 0), (0, 0), (0, 128 - na)))

    out = pl.pallas_call(
        functools.partial(_loss_kernel, nchunk=nchunk, chunk_r=chunk_r),
        grid=(b, 2, nchunk),
        in_specs=[
            pl.BlockSpec((1, 1, 128), lambda bb, p, c: (bb, 0, 0)),
            pl.BlockSpec((1, 1, 128), lambda bb, p, c: (bb, 0, 0)),
            pl.BlockSpec((1, chunk_r, _LANES), lambda bb, p, c: (bb, c, 0)),
            pl.BlockSpec((1, chunk_r, _LANES), lambda bb, p, c: (bb, c, 0)),
        ],
        out_specs=pl.BlockSpec((1, 1, 128), lambda bb, p, c: (bb, 0, 0)),
        out_shape=jax.ShapeDtypeStruct((b, 1, 128), jnp.float32),
        scratch_shapes=[
            pltpu.SMEM((4,), jnp.float32),
            pltpu.VMEM((_BINS, _BINS), jnp.float32),
            pltpu.VMEM((_BINS, _SUB_R, _LANES), jnp.bfloat16),
            pltpu.VMEM((_BINS, _SUB_R, _LANES), jnp.bfloat16),
        ],
        compiler_params=pltpu.CompilerParams(
            dimension_semantics=("parallel", "arbitrary", "arbitrary")),
    )(pa, ta, f, w)

    affine_loss = jnp.sum(out[:, 0, 0]) / (b * na)
    sim_loss = -jnp.mean(out[:, 0, 1])
    return affine_loss + sim_loss


# speedup vs baseline: 6.5241x; 6.5241x over previous
import functools

import jax
import jax.numpy as jnp
from jax.experimental import pallas as pl
from jax.experimental.pallas import tpu as pltpu

_BINS = 64
_EPS = 1e-10
_LANES = 1024
_SUB_R = 16


def _loss_kernel(pa_ref, ta_ref, f_ref, w_ref, out_ref, mm_ref, acc_ref,
                 *, nchunk, chunk_r):
    phase = pl.program_id(1)
    c = pl.program_id(2)

    @pl.when(jnp.logical_and(phase == 0, c == 0))
    def _init_mm():
        mm_ref[0] = jnp.float32(jnp.inf)
        mm_ref[1] = jnp.float32(-jnp.inf)
        mm_ref[2] = jnp.float32(jnp.inf)
        mm_ref[3] = jnp.float32(-jnp.inf)

    @pl.when(phase == 0)
    def _minmax():
        fb = f_ref[0]
        wb = w_ref[0]
        mm_ref[0] = jnp.minimum(mm_ref[0], jnp.min(fb))
        mm_ref[1] = jnp.maximum(mm_ref[1], jnp.max(fb))
        mm_ref[2] = jnp.minimum(mm_ref[2], jnp.min(wb))
        mm_ref[3] = jnp.maximum(mm_ref[3], jnp.max(wb))

    @pl.when(phase == 1)
    def _hist():
        @pl.when(c == 0)
        def _zero_acc():
            acc_ref[...] = jnp.zeros_like(acc_ref)

        fmin = mm_ref[0]
        wmin = mm_ref[2]
        inv_f = 1.0 / (mm_ref[1] - fmin + _EPS)
        inv_w = 1.0 / (mm_ref[3] - wmin + _EPS)

        bins2d = jax.lax.broadcasted_iota(
            jnp.int32, (_BINS, _LANES), 0).astype(jnp.int8)
        f8t = jnp.float8_e4m3fn

        def body(i, _):
            f8 = f_ref[0, pl.ds(i * _SUB_R, _SUB_R), :]
            w8 = w_ref[0, pl.ds(i * _SUB_R, _SUB_R), :]
            fi8 = jnp.clip(jnp.floor((f8 - fmin) * inv_f * (_BINS - 1)),
                           0.0, _BINS - 1.0).astype(jnp.int8)
            wi8 = jnp.clip(jnp.floor((w8 - wmin) * inv_w * (_BINS - 1)),
                           0.0, _BINS - 1.0).astype(jnp.int8)
            parts = []
            for j in range(_SUB_R):
                fr = jax.lax.slice(fi8, (j, 0), (j + 1, _LANES))
                wr = jax.lax.slice(wi8, (j, 0), (j + 1, _LANES))
                ohf = jnp.where(bins2d == fr, f8t(1), f8t(0))
                ohw = jnp.where(bins2d == wr, f8t(1), f8t(0))
                parts.append(jax.lax.dot_general(
                    ohf, ohw, (((1,), (1,)), ((), ())),
                    preferred_element_type=jnp.float32))
            while len(parts) > 1:
                parts = [a + b for a, b in zip(parts[::2], parts[1::2])]
            acc_ref[...] += parts[0]
            return 0

        jax.lax.fori_loop(0, chunk_r // _SUB_R, body, 0)

        @pl.when(c == nchunk - 1)
        def _finalize():
            counts = acc_ref[...]
            total = jnp.sum(counts)
            inv_n = 1.0 / (total + _EPS)
            joint = counts * inv_n
            p_f = jnp.sum(joint, axis=1, keepdims=True)
            p_w = jnp.sum(joint, axis=0, keepdims=True)
            h_f = -jnp.sum(p_f * jnp.log(p_f + _EPS))
            h_w = -jnp.sum(p_w * jnp.log(p_w + _EPS))
            h_j = -jnp.sum(joint * jnp.log(joint + _EPS))
            mi = h_f + h_w - h_j
            nmi = 2.0 * mi / (h_f + h_w + _EPS)
            d = pa_ref[0] - ta_ref[0]
            sumsq = jnp.sum(d * d)
            lane = jax.lax.broadcasted_iota(jnp.int32, (1, 128), 1)
            out_ref[0] = jnp.where(lane == 0, sumsq,
                                   jnp.where(lane == 1, nmi, 0.0))


def _choose_chunk(nr):
    for cr in (128, 64, 32, 16, 8):
        if nr % cr == 0:
            return cr
    return nr


@jax.jit
def kernel(predicted_affine, true_affine, fixed, warped):
    b = fixed.shape[0]
    n = fixed.shape[2] * fixed.shape[3] * fixed.shape[4]
    nr = n // _LANES
    chunk_r = _choose_chunk(nr)
    nchunk = nr // chunk_r

    f = fixed[:, 0].reshape(b, nr, _LANES)
    w = warped[:, 0].reshape(b, nr, _LANES)
    na = predicted_affine.shape[1]
    pa = jnp.pad(predicted_affine[:, None, :], ((0, 0), (0, 0), (0, 128 - na)))
    ta = jnp.pad(true_affine[:, None, :], ((0, 0), (0, 0), (0, 128 - na)))

    out = pl.pallas_call(
        functools.partial(_loss_kernel, nchunk=nchunk, chunk_r=chunk_r),
        grid=(b, 2, nchunk),
        in_specs=[
            pl.BlockSpec((1, 1, 128), lambda bb, p, c: (bb, 0, 0)),
            pl.BlockSpec((1, 1, 128), lambda bb, p, c: (bb, 0, 0)),
            pl.BlockSpec((1, chunk_r, _LANES), lambda bb, p, c: (bb, c, 0)),
            pl.BlockSpec((1, chunk_r, _LANES), lambda bb, p, c: (bb, c, 0)),
        ],
        out_specs=pl.BlockSpec((1, 1, 128), lambda bb, p, c: (bb, 0, 0)),
        out_shape=jax.ShapeDtypeStruct((b, 1, 128), jnp.float32),
        scratch_shapes=[
            pltpu.SMEM((4,), jnp.float32),
            pltpu.VMEM((_BINS, _BINS), jnp.float32),
        ],
        compiler_params=pltpu.CompilerParams(
            dimension_semantics=("parallel", "arbitrary", "arbitrary")),
    )(pa, ta, f, w)

    affine_loss = jnp.sum(out[:, 0, 0]) / (b * na)
    sim_loss = -jnp.mean(out[:, 0, 1])
    return affine_loss + sim_loss


# speedup vs baseline: 6.6961x; 1.0264x over previous
import functools

import jax
import jax.numpy as jnp
from jax.experimental import pallas as pl
from jax.experimental.pallas import tpu as pltpu

_BINS = 64
_EPS = 1e-10
_LANES = 1024
_SUB_R = 16


def _loss_kernel(pa_ref, ta_ref, f_ref, w_ref, out_ref, mm_ref, acc_ref,
                 *, nchunk, chunk_r):
    phase = pl.program_id(1)
    c = pl.program_id(2)

    @pl.when(jnp.logical_and(phase == 0, c == 0))
    def _init_mm():
        mm_ref[0] = jnp.float32(jnp.inf)
        mm_ref[1] = jnp.float32(-jnp.inf)
        mm_ref[2] = jnp.float32(jnp.inf)
        mm_ref[3] = jnp.float32(-jnp.inf)

    @pl.when(phase == 0)
    def _minmax():
        fb = f_ref[0]
        wb = w_ref[0]
        mm_ref[0] = jnp.minimum(mm_ref[0], jnp.min(fb))
        mm_ref[1] = jnp.maximum(mm_ref[1], jnp.max(fb))
        mm_ref[2] = jnp.minimum(mm_ref[2], jnp.min(wb))
        mm_ref[3] = jnp.maximum(mm_ref[3], jnp.max(wb))

    @pl.when(phase == 1)
    def _hist():
        @pl.when(c == 0)
        def _zero_acc():
            acc_ref[...] = jnp.zeros_like(acc_ref)

        fmin = mm_ref[0]
        wmin = mm_ref[2]
        inv_f = 1.0 / (mm_ref[1] - fmin + _EPS)
        inv_w = 1.0 / (mm_ref[3] - wmin + _EPS)

        bins2d = jax.lax.broadcasted_iota(
            jnp.int32, (_BINS, _LANES), 0).astype(jnp.int8)
        f8t = jnp.float8_e4m3fn

        def body(i, _):
            f8 = f_ref[0, pl.ds(i * _SUB_R, _SUB_R), :]
            w8 = w_ref[0, pl.ds(i * _SUB_R, _SUB_R), :]
            fi8 = jnp.clip(jnp.floor((f8 - fmin) * inv_f * (_BINS - 1)),
                           0.0, _BINS - 1.0).astype(jnp.int8)
            wi8 = jnp.clip(jnp.floor((w8 - wmin) * inv_w * (_BINS - 1)),
                           0.0, _BINS - 1.0).astype(jnp.int8)
            acc = acc_ref[...]
            for j in range(_SUB_R):
                fr = jax.lax.slice(fi8, (j, 0), (j + 1, _LANES))
                wr = jax.lax.slice(wi8, (j, 0), (j + 1, _LANES))
                ohf = jnp.where(bins2d == fr, f8t(1), f8t(0))
                ohw = jnp.where(bins2d == wr, f8t(1), f8t(0))
                acc = jax.lax.dot_general(
                    ohf, ohw, (((1,), (1,)), ((), ())),
                    preferred_element_type=jnp.float32) + acc
            acc_ref[...] = acc
            return 0

        jax.lax.fori_loop(0, chunk_r // _SUB_R, body, 0)

        @pl.when(c == nchunk - 1)
        def _finalize():
            counts = acc_ref[...]
            total = jnp.sum(counts)
            inv_n = 1.0 / (total + _EPS)
            joint = counts * inv_n
            p_f = jnp.sum(joint, axis=1, keepdims=True)
            p_w = jnp.sum(joint, axis=0, keepdims=True)
            h_f = -jnp.sum(p_f * jnp.log(p_f + _EPS))
            h_w = -jnp.sum(p_w * jnp.log(p_w + _EPS))
            h_j = -jnp.sum(joint * jnp.log(joint + _EPS))
            mi = h_f + h_w - h_j
            nmi = 2.0 * mi / (h_f + h_w + _EPS)
            d = pa_ref[0] - ta_ref[0]
            sumsq = jnp.sum(d * d)
            lane = jax.lax.broadcasted_iota(jnp.int32, (1, 128), 1)
            out_ref[0] = jnp.where(lane == 0, sumsq,
                                   jnp.where(lane == 1, nmi, 0.0))


def _choose_chunk(nr):
    for cr in (128, 64, 32, 16, 8):
        if nr % cr == 0:
            return cr
    return nr


@jax.jit
def kernel(predicted_affine, true_affine, fixed, warped):
    b = fixed.shape[0]
    n = fixed.shape[2] * fixed.shape[3] * fixed.shape[4]
    nr = n // _LANES
    chunk_r = _choose_chunk(nr)
    nchunk = nr // chunk_r

    f = fixed[:, 0].reshape(b, nr, _LANES)
    w = warped[:, 0].reshape(b, nr, _LANES)
    na = predicted_affine.shape[1]
    pa = jnp.pad(predicted_affine[:, None, :], ((0, 0), (0, 0), (0, 128 - na)))
    ta = jnp.pad(true_affine[:, None, :], ((0, 0), (0, 0), (0, 128 - na)))

    out = pl.pallas_call(
        functools.partial(_loss_kernel, nchunk=nchunk, chunk_r=chunk_r),
        grid=(b, 2, nchunk),
        in_specs=[
            pl.BlockSpec((1, 1, 128), lambda bb, p, c: (bb, 0, 0)),
            pl.BlockSpec((1, 1, 128), lambda bb, p, c: (bb, 0, 0)),
            pl.BlockSpec((1, chunk_r, _LANES), lambda bb, p, c: (bb, c, 0)),
            pl.BlockSpec((1, chunk_r, _LANES), lambda bb, p, c: (bb, c, 0)),
        ],
        out_specs=pl.BlockSpec((1, 1, 128), lambda bb, p, c: (bb, 0, 0)),
        out_shape=jax.ShapeDtypeStruct((b, 1, 128), jnp.float32),
        scratch_shapes=[
            pltpu.SMEM((4,), jnp.float32),
            pltpu.VMEM((_BINS, _BINS), jnp.float32),
        ],
        compiler_params=pltpu.CompilerParams(
            dimension_semantics=("parallel", "arbitrary", "arbitrary")),
    )(pa, ta, f, w)

    affine_loss = jnp.sum(out[:, 0, 0]) / (b * na)
    sim_loss = -jnp.mean(out[:, 0, 1])
    return affine_loss + sim_loss
